# baseline (speedup 1.0000x reference)
import jax
import jax.numpy as jnp
from jax import lax
from jax.experimental import pallas as pl
from jax.experimental.pallas import tpu as pltpu
from jax.experimental.pallas import tpu_sc as plsc

N = 10000
E = 320000
G = 64
D = 128
DH = 64
D_LAT = 64

NP_ = 10240
NC = 2
NS = 16
CH = 80
EPW = E // NS
CPW = EPW // CH
TPN = NP_ // NS
NB = 3

RB = NP_
NBLK = NP_ // RB


def _sc_edge_body(src_hbm, dst_hbm, h2_hbm, aux_hbm,
                  numer_out, denom_out,
                  as_v, ad_v, src_v, dst_v, sadj_v, rows_v, w_v,
                  sem_g, sem_s, numer_s, denom_s):
    c = lax.axis_index("c")
    s = lax.axis_index("s")

    pltpu.sync_copy(aux_hbm.at[0], as_v)
    pltpu.sync_copy(aux_hbm.at[1], ad_v)
    pltpu.sync_copy(src_hbm.at[s], src_v)
    pltpu.sync_copy(dst_hbm.at[s], dst_v)

    hbase = c * NP_

    def compute_w(jj, b):
        for k in range(CH // 16):
            si = src_v[jj, pl.ds(k * 16, 16)]
            di = dst_v[jj, pl.ds(k * 16, 16)]
            sadj_v[b, pl.ds(k * 16, 16)] = si + hbase
            z = plsc.load_gather(as_v, [si]) + plsc.load_gather(ad_v, [di])
            z = jnp.where(z >= 0.0, z, 0.2 * z)
            w_v[b, pl.ds(k * 16, 16)] = jnp.exp(z)

    def gather_desc(b):
        return pltpu.make_async_copy(h2_hbm.at[sadj_v.at[b]], rows_v.at[b],
                                     sem_g.at[b])

    def scatter_descs(jj, b):
        return (pltpu.make_async_copy(rows_v.at[b], numer_s.at[dst_v.at[jj]],
                                      sem_s.at[b]),
                pltpu.make_async_copy(w_v.at[b], denom_s.at[dst_v.at[jj]],
                                      sem_s.at[b]))

    for b in range(NB - 1):
        compute_w(b, b)
        gather_desc(b).start()

    zeros16 = jnp.zeros((16,), jnp.float32)

    def _zrow(r, _):
        for cc in range(DH // 16):
            rows_v[NB - 1, r, pl.ds(cc * 16, 16)] = zeros16
        return 0

    lax.fori_loop(0, CH, _zrow, 0)
    for k in range(CH // 16):
        w_v[NB - 1, pl.ds(k * 16, 16)] = zeros16
    for k in range(TPN // CH):
        pltpu.sync_copy(rows_v.at[NB - 1],
                        numer_s.at[pl.ds(s * TPN + k * CH, CH)])
        pltpu.sync_copy(w_v.at[NB - 1],
                        denom_s.at[pl.ds(s * TPN + k * CH, CH)])
    plsc.subcore_barrier()

    def chunk_body(j, _):
        p = lax.rem(j, NB)
        r = lax.rem(j + NB - 1, NB)

        @pl.when(j + NB - 1 < CPW)
        def _():
            @pl.when(j >= 1)
            def _():
                for d in scatter_descs(j - 1, r):
                    d.wait()
            compute_w(j + NB - 1, r)
            gather_desc(r).start()

        gather_desc(p).wait()

        for g in range(CH // 16):
            w16 = w_v[p, pl.ds(g * 16, 16)]
            for i in range(16):
                rr = g * 16 + i
                ws = w16[i]
                for cc in range(DH // 16):
                    rows_v[p, rr, pl.ds(cc * 16, 16)] = (
                        rows_v[p, rr, pl.ds(cc * 16, 16)] * ws)

        pltpu.async_copy(rows_v.at[p], numer_s.at[dst_v.at[j]], sem_s.at[p],
                         add=True)
        pltpu.async_copy(w_v.at[p], denom_s.at[dst_v.at[j]], sem_s.at[p],
                         add=True)
        return 0

    lax.fori_loop(0, CPW, chunk_body, 0)
    for j in range(CPW - NB, CPW):
        for d in scatter_descs(j, j % NB):
            d.wait()
    plsc.subcore_barrier()

    pltpu.sync_copy(numer_s.at[pl.ds(s * TPN, TPN)],
                    numer_out.at[c, pl.ds(s * TPN, TPN)])
    pltpu.sync_copy(denom_s.at[pl.ds(s * TPN, TPN)],
                    denom_out.at[c, pl.ds(s * TPN, TPN)])


def _sc_edge_pass(src3d, dst3d, h2, aux):
    numer = jax.ShapeDtypeStruct((NC, NP_, DH), jnp.float32)
    denom = jax.ShapeDtypeStruct((NC, NP_), jnp.float32)
    f = pl.kernel(
        _sc_edge_body,
        out_type=(numer, denom),
        mesh=plsc.VectorSubcoreMesh(core_axis_name="c", subcore_axis_name="s",
                                    num_cores=NC, num_subcores=NS),
        compiler_params=pltpu.CompilerParams(needs_layout_passes=False,
                                             use_tc_tiling_on_sc=False),
        scratch_types=[
            pltpu.VMEM((NP_,), jnp.float32),
            pltpu.VMEM((NP_,), jnp.float32),
            pltpu.VMEM((CPW, CH), jnp.int32),
            pltpu.VMEM((CPW, CH), jnp.int32),
            pltpu.VMEM((NB, CH), jnp.int32),
            pltpu.VMEM((NB, CH, DH), jnp.float32),
            pltpu.VMEM((NB, CH), jnp.float32),
            pltpu.SemaphoreType.DMA((NB,)),
            pltpu.SemaphoreType.DMA((NB,)),
            pltpu.VMEM_SHARED((NP_, DH), jnp.float32),
            pltpu.VMEM_SHARED((NP_,), jnp.float32),
        ],
    )
    return f(src3d, dst3d, h2, aux)


def _split(h):
    return jnp.stack([h[:, :DH], h[:, DH:]], axis=0)


def _aux_of(h, a_src_ref, a_dst_ref):
    a2 = jnp.concatenate([a_src_ref[...][:, None], a_dst_ref[...][:, None]],
                         axis=1)
    return jnp.dot(h, a2, preferred_element_type=jnp.float32).T


def _tc_entry_body(x_ref, w_ref, as_ref, ad_ref, h2_ref, aux_ref):
    h = jnp.dot(x_ref[...], w_ref[...], preferred_element_type=jnp.float32)
    h2_ref[...] = _split(h)
    aux_ref[...] = _aux_of(h, as_ref, ad_ref)


def _tc_entry(x, w, a_src, a_dst):
    return pl.pallas_call(
        _tc_entry_body,
        grid=(NBLK,),
        in_specs=[
            pl.BlockSpec((RB, D), lambda i: (i, 0)),
            pl.BlockSpec((D, D), lambda i: (0, 0)),
            pl.BlockSpec((D,), lambda i: (0,)),
            pl.BlockSpec((D,), lambda i: (0,)),
        ],
        out_specs=[
            pl.BlockSpec((NC, RB, DH), lambda i: (0, i, 0)),
            pl.BlockSpec((2, RB), lambda i: (0, i)),
        ],
        out_shape=[
            jax.ShapeDtypeStruct((NC, NP_, DH), jnp.float32),
            jax.ShapeDtypeStruct((2, NP_), jnp.float32),
        ],
    )(x, w, a_src, a_dst)


def _epilogue_block(numer_ref, denom_ref, h2_ref, aux_ref, b_ref):
    aux = aux_ref[...]
    z = (aux[0, :] + aux[1, :])[:, None]
    wself = jnp.exp(jnp.where(z >= 0.0, z, 0.2 * z))
    h = jnp.concatenate([h2_ref[0], h2_ref[1]], axis=-1)
    numer = jnp.concatenate([numer_ref[0], numer_ref[1]], axis=-1) + wself * h
    denom = denom_ref[0, :] + wself[:, 0] + 1e-16
    return numer / denom[:, None] + b_ref[...]


def _tc_mid_body(numer_ref, denom_ref, h2_ref, aux_ref, b_ref, w_ref,
                 as_ref, ad_ref, h2o_ref, aux2_ref):
    x = _epilogue_block(numer_ref, denom_ref, h2_ref, aux_ref, b_ref)
    h2 = jnp.dot(x, w_ref[...], preferred_element_type=jnp.float32)
    h2o_ref[...] = _split(h2)
    aux2_ref[...] = _aux_of(h2, as_ref, ad_ref)


def _tc_mid(numer, denom, h2, aux, b, w, a_src, a_dst):
    return pl.pallas_call(
        _tc_mid_body,
        grid=(NBLK,),
        in_specs=[
            pl.BlockSpec((NC, RB, DH), lambda i: (0, i, 0)),
            pl.BlockSpec((NC, RB), lambda i: (0, i)),
            pl.BlockSpec((NC, RB, DH), lambda i: (0, i, 0)),
            pl.BlockSpec((2, RB), lambda i: (0, i)),
            pl.BlockSpec((1, D), lambda i: (0, 0)),
            pl.BlockSpec((D, D), lambda i: (0, 0)),
            pl.BlockSpec((D,), lambda i: (0,)),
            pl.BlockSpec((D,), lambda i: (0,)),
        ],
        out_specs=[
            pl.BlockSpec((NC, RB, DH), lambda i: (0, i, 0)),
            pl.BlockSpec((2, RB), lambda i: (0, i)),
        ],
        out_shape=[
            jax.ShapeDtypeStruct((NC, NP_, DH), jnp.float32),
            jax.ShapeDtypeStruct((2, NP_), jnp.float32),
        ],
    )(numer, denom, h2, aux, b, w, a_src, a_dst)


def _tc_final_body(numer_ref, denom_ref, h2_ref, aux_ref, b_ref, batch_ref,
                   bn_gamma_ref, bn_beta_ref, bn_rm_ref, bn_rv_ref,
                   fcw_ref, fcb_ref, out_ref):
    x = _epilogue_block(numer_ref, denom_ref, h2_ref, aux_ref, b_ref)
    x = jnp.where(lax.broadcasted_iota(jnp.int32, (RB, D), 0) < N, x, 0.0)
    onehot = (lax.broadcasted_iota(jnp.int32, (G, RB), 0)
              == batch_ref[...][None, :]).astype(jnp.float32)
    pooled = jnp.dot(onehot, x, preferred_element_type=jnp.float32)
    bn_scale = (bn_gamma_ref[...] / jnp.sqrt(bn_rv_ref[...] + 1e-5))[None, :]
    bn_shift = (bn_beta_ref[...] - bn_rm_ref[...]
                / jnp.sqrt(bn_rv_ref[...] + 1e-5)
                * bn_gamma_ref[...])[None, :]
    normed = pooled * bn_scale + bn_shift
    out_ref[...] = lax.dot_general(
        normed, fcw_ref[...], (((1,), (1,)), ((), ())),
        preferred_element_type=jnp.float32) + fcb_ref[...][None, :]


def _tc_final(numer, denom, h2, aux, b, batch,
              bn_gamma, bn_beta, bn_rm, bn_rv, fc_W, fc_b):
    return pl.pallas_call(
        _tc_final_body,
        grid=(NBLK,),
        in_specs=[
            pl.BlockSpec((NC, RB, DH), lambda i: (0, i, 0)),
            pl.BlockSpec((NC, RB), lambda i: (0, i)),
            pl.BlockSpec((NC, RB, DH), lambda i: (0, i, 0)),
            pl.BlockSpec((2, RB), lambda i: (0, i)),
            pl.BlockSpec((1, D), lambda i: (0, 0)),
            pl.BlockSpec((RB,), lambda i: (i,)),
            pl.BlockSpec((D,), lambda i: (0,)),
            pl.BlockSpec((D,), lambda i: (0,)),
            pl.BlockSpec((D,), lambda i: (0,)),
            pl.BlockSpec((D,), lambda i: (0,)),
            pl.BlockSpec((D_LAT, D), lambda i: (0, 0)),
            pl.BlockSpec((D_LAT,), lambda i: (0,)),
        ],
        out_specs=pl.BlockSpec((G, D_LAT), lambda i: (0, 0)),
        out_shape=jax.ShapeDtypeStruct((G, D_LAT), jnp.float32),
    )(numer, denom, h2, aux, b, batch,
      bn_gamma, bn_beta, bn_rm, bn_rv, fc_W, fc_b)


def kernel(x, edge_index, batch, W0, a_src0, a_dst0, b0, W1, a_src1, a_dst1,
           b1, W2, a_src2, a_dst2, b2, bn_gamma, bn_beta, bn_rm, bn_rv,
           fc_W, fc_b):
    src3d = edge_index[0].reshape(NS, CPW, CH)
    dst3d = edge_index[1].reshape(NS, CPW, CH)

    ws = [(W0, a_src0, a_dst0, b0.reshape(1, D)),
          (W1, a_src1, a_dst1, b1.reshape(1, D)),
          (W2, a_src2, a_dst2, b2.reshape(1, D))]

    h2, aux = _tc_entry(x, ws[0][0], ws[0][1], ws[0][2])
    for l in (1, 2):
        numer, denom = _sc_edge_pass(src3d, dst3d,
                                     h2.reshape(NC * NP_, DH), aux)
        h2, aux = _tc_mid(numer, denom, h2, aux, ws[l - 1][3],
                          ws[l][0], ws[l][1], ws[l][2])
    numer, denom = _sc_edge_pass(src3d, dst3d, h2.reshape(NC * NP_, DH), aux)

    return _tc_final(numer, denom, h2, aux, ws[2][3], batch,
                     bn_gamma, bn_beta, bn_rm, bn_rv, fc_W, fc_b)


# speedup vs baseline: 1.0181x; 1.0181x over previous
import jax
import jax.numpy as jnp
from jax import lax
from jax.experimental import pallas as pl
from jax.experimental.pallas import tpu as pltpu
from jax.experimental.pallas import tpu_sc as plsc

N = 10000
E = 320000
G = 64
D = 128
DH = 64
D_LAT = 64

NP_ = 10240
NC = 2
NS = 16
CH = 80
EPW = E // NS
CPW = EPW // CH
TPN = NP_ // NS
NB = 3

RB = 2048
NBLK = NP_ // RB


def _sc_edge_body(src_hbm, dst_hbm, h2_hbm, aux_hbm,
                  numer_out, denom_out,
                  as_v, ad_v, src_v, dst_v, sadj_v, rows_v, w_v,
                  sem_g, sem_s, numer_s, denom_s):
    c = lax.axis_index("c")
    s = lax.axis_index("s")

    pltpu.sync_copy(aux_hbm.at[0], as_v)
    pltpu.sync_copy(aux_hbm.at[1], ad_v)
    pltpu.sync_copy(src_hbm.at[s], src_v)
    pltpu.sync_copy(dst_hbm.at[s], dst_v)

    hbase = c * NP_

    def compute_w(jj, b):
        for k in range(CH // 16):
            si = src_v[jj, pl.ds(k * 16, 16)]
            di = dst_v[jj, pl.ds(k * 16, 16)]
            sadj_v[b, pl.ds(k * 16, 16)] = si + hbase
            z = plsc.load_gather(as_v, [si]) + plsc.load_gather(ad_v, [di])
            z = jnp.where(z >= 0.0, z, 0.2 * z)
            w_v[b, pl.ds(k * 16, 16)] = jnp.exp(z)

    def gather_desc(b):
        return pltpu.make_async_copy(h2_hbm.at[sadj_v.at[b]], rows_v.at[b],
                                     sem_g.at[b])

    def scatter_descs(jj, b):
        return (pltpu.make_async_copy(rows_v.at[b], numer_s.at[dst_v.at[jj]],
                                      sem_s.at[b]),
                pltpu.make_async_copy(w_v.at[b], denom_s.at[dst_v.at[jj]],
                                      sem_s.at[b]))

    for b in range(NB - 1):
        compute_w(b, b)
        gather_desc(b).start()

    zeros16 = jnp.zeros((16,), jnp.float32)

    def _zrow(r, _):
        for cc in range(DH // 16):
            rows_v[NB - 1, r, pl.ds(cc * 16, 16)] = zeros16
        return 0

    lax.fori_loop(0, CH, _zrow, 0)
    for k in range(CH // 16):
        w_v[NB - 1, pl.ds(k * 16, 16)] = zeros16
    for k in range(TPN // CH):
        pltpu.sync_copy(rows_v.at[NB - 1],
                        numer_s.at[pl.ds(s * TPN + k * CH, CH)])
        pltpu.sync_copy(w_v.at[NB - 1],
                        denom_s.at[pl.ds(s * TPN + k * CH, CH)])
    plsc.subcore_barrier()

    def chunk_body(j, _):
        p = lax.rem(j, NB)
        r = lax.rem(j + NB - 1, NB)

        @pl.when(j + NB - 1 < CPW)
        def _():
            @pl.when(j >= 1)
            def _():
                for d in scatter_descs(j - 1, r):
                    d.wait()
            compute_w(j + NB - 1, r)
            gather_desc(r).start()

        gather_desc(p).wait()

        for g in range(CH // 16):
            w16 = w_v[p, pl.ds(g * 16, 16)]
            for i in range(16):
                rr = g * 16 + i
                ws = w16[i]
                for cc in range(DH // 16):
                    rows_v[p, rr, pl.ds(cc * 16, 16)] = (
                        rows_v[p, rr, pl.ds(cc * 16, 16)] * ws)

        pltpu.async_copy(rows_v.at[p], numer_s.at[dst_v.at[j]], sem_s.at[p],
                         add=True)
        pltpu.async_copy(w_v.at[p], denom_s.at[dst_v.at[j]], sem_s.at[p],
                         add=True)
        return 0

    lax.fori_loop(0, CPW, chunk_body, 0)
    for j in range(CPW - NB, CPW):
        for d in scatter_descs(j, j % NB):
            d.wait()
    plsc.subcore_barrier()

    pltpu.sync_copy(numer_s.at[pl.ds(s * TPN, TPN)],
                    numer_out.at[c, pl.ds(s * TPN, TPN)])
    pltpu.sync_copy(denom_s.at[pl.ds(s * TPN, TPN)],
                    denom_out.at[c, pl.ds(s * TPN, TPN)])


def _sc_edge_pass(src3d, dst3d, h2, aux):
    numer = jax.ShapeDtypeStruct((NC, NP_, DH), jnp.float32)
    denom = jax.ShapeDtypeStruct((NC, NP_), jnp.float32)
    f = pl.kernel(
        _sc_edge_body,
        out_type=(numer, denom),
        mesh=plsc.VectorSubcoreMesh(core_axis_name="c", subcore_axis_name="s",
                                    num_cores=NC, num_subcores=NS),
        compiler_params=pltpu.CompilerParams(needs_layout_passes=False,
                                             use_tc_tiling_on_sc=False),
        scratch_types=[
            pltpu.VMEM((NP_,), jnp.float32),
            pltpu.VMEM((NP_,), jnp.float32),
            pltpu.VMEM((CPW, CH), jnp.int32),
            pltpu.VMEM((CPW, CH), jnp.int32),
            pltpu.VMEM((NB, CH), jnp.int32),
            pltpu.VMEM((NB, CH, DH), jnp.float32),
            pltpu.VMEM((NB, CH), jnp.float32),
            pltpu.SemaphoreType.DMA((NB,)),
            pltpu.SemaphoreType.DMA((NB,)),
            pltpu.VMEM_SHARED((NP_, DH), jnp.float32),
            pltpu.VMEM_SHARED((NP_,), jnp.float32),
        ],
    )
    return f(src3d, dst3d, h2, aux)


def _split(h):
    return jnp.stack([h[:, :DH], h[:, DH:]], axis=0)


def _aux_of(h, a_src_ref, a_dst_ref):
    a2 = jnp.concatenate([a_src_ref[...][:, None], a_dst_ref[...][:, None]],
                         axis=1)
    return jnp.dot(h, a2, preferred_element_type=jnp.float32).T


def _tc_entry_body(x_ref, w_ref, as_ref, ad_ref, h2_ref, aux_ref):
    h = jnp.dot(x_ref[...], w_ref[...], preferred_element_type=jnp.float32)
    h2_ref[...] = _split(h)
    aux_ref[...] = _aux_of(h, as_ref, ad_ref)


def _tc_entry(x, w, a_src, a_dst):
    return pl.pallas_call(
        _tc_entry_body,
        grid=(NBLK,),
        in_specs=[
            pl.BlockSpec((RB, D), lambda i: (i, 0)),
            pl.BlockSpec((D, D), lambda i: (0, 0)),
            pl.BlockSpec((D,), lambda i: (0,)),
            pl.BlockSpec((D,), lambda i: (0,)),
        ],
        out_specs=[
            pl.BlockSpec((NC, RB, DH), lambda i: (0, i, 0)),
            pl.BlockSpec((2, RB), lambda i: (0, i)),
        ],
        out_shape=[
            jax.ShapeDtypeStruct((NC, NP_, DH), jnp.float32),
            jax.ShapeDtypeStruct((2, NP_), jnp.float32),
        ],
    )(x, w, a_src, a_dst)


def _epilogue_block(numer_ref, denom_ref, h2_ref, aux_ref, b_ref):
    aux = aux_ref[...]
    z = (aux[0, :] + aux[1, :])[:, None]
    wself = jnp.exp(jnp.where(z >= 0.0, z, 0.2 * z))
    h = jnp.concatenate([h2_ref[0], h2_ref[1]], axis=-1)
    numer = jnp.concatenate([numer_ref[0], numer_ref[1]], axis=-1) + wself * h
    denom = denom_ref[0, :] + wself[:, 0] + 1e-16
    return numer / denom[:, None] + b_ref[...]


def _tc_mid_body(numer_ref, denom_ref, h2_ref, aux_ref, b_ref, w_ref,
                 as_ref, ad_ref, h2o_ref, aux2_ref):
    x = _epilogue_block(numer_ref, denom_ref, h2_ref, aux_ref, b_ref)
    h2 = jnp.dot(x, w_ref[...], preferred_element_type=jnp.float32)
    h2o_ref[...] = _split(h2)
    aux2_ref[...] = _aux_of(h2, as_ref, ad_ref)


def _tc_mid(numer, denom, h2, aux, b, w, a_src, a_dst):
    return pl.pallas_call(
        _tc_mid_body,
        grid=(NBLK,),
        in_specs=[
            pl.BlockSpec((NC, RB, DH), lambda i: (0, i, 0)),
            pl.BlockSpec((NC, RB), lambda i: (0, i)),
            pl.BlockSpec((NC, RB, DH), lambda i: (0, i, 0)),
            pl.BlockSpec((2, RB), lambda i: (0, i)),
            pl.BlockSpec((1, D), lambda i: (0, 0)),
            pl.BlockSpec((D, D), lambda i: (0, 0)),
            pl.BlockSpec((D,), lambda i: (0,)),
            pl.BlockSpec((D,), lambda i: (0,)),
        ],
        out_specs=[
            pl.BlockSpec((NC, RB, DH), lambda i: (0, i, 0)),
            pl.BlockSpec((2, RB), lambda i: (0, i)),
        ],
        out_shape=[
            jax.ShapeDtypeStruct((NC, NP_, DH), jnp.float32),
            jax.ShapeDtypeStruct((2, NP_), jnp.float32),
        ],
    )(numer, denom, h2, aux, b, w, a_src, a_dst)


def _tc_final_body(numer_ref, denom_ref, h2_ref, aux_ref, b_ref, batch_ref,
                   bn_gamma_ref, bn_beta_ref, bn_rm_ref, bn_rv_ref,
                   fcw_ref, fcb_ref, out_ref, pooled_ref):
    i = pl.program_id(0)
    x = _epilogue_block(numer_ref, denom_ref, h2_ref, aux_ref, b_ref)
    onehot = (lax.broadcasted_iota(jnp.int32, (G, RB), 0)
              == batch_ref[...][None, :]).astype(jnp.float32)
    part = jnp.dot(onehot, x, preferred_element_type=jnp.float32)

    @pl.when(i == 0)
    def _():
        pooled_ref[...] = jnp.zeros_like(pooled_ref)

    pooled_ref[...] += part

    @pl.when(i == NBLK - 1)
    def _():
        bn_scale = (bn_gamma_ref[...]
                    / jnp.sqrt(bn_rv_ref[...] + 1e-5))[None, :]
        bn_shift = (bn_beta_ref[...] - bn_rm_ref[...]
                    / jnp.sqrt(bn_rv_ref[...] + 1e-5)
                    * bn_gamma_ref[...])[None, :]
        normed = pooled_ref[...] * bn_scale + bn_shift
        out_ref[...] = lax.dot_general(
            normed, fcw_ref[...], (((1,), (1,)), ((), ())),
            preferred_element_type=jnp.float32) + fcb_ref[...][None, :]


def _tc_final(numer, denom, h2, aux, b, batch,
              bn_gamma, bn_beta, bn_rm, bn_rv, fc_W, fc_b):
    out, _ = pl.pallas_call(
        _tc_final_body,
        grid=(NBLK,),
        in_specs=[
            pl.BlockSpec((NC, RB, DH), lambda i: (0, i, 0)),
            pl.BlockSpec((NC, RB), lambda i: (0, i)),
            pl.BlockSpec((NC, RB, DH), lambda i: (0, i, 0)),
            pl.BlockSpec((2, RB), lambda i: (0, i)),
            pl.BlockSpec((1, D), lambda i: (0, 0)),
            pl.BlockSpec((RB,), lambda i: (i,)),
            pl.BlockSpec((D,), lambda i: (0,)),
            pl.BlockSpec((D,), lambda i: (0,)),
            pl.BlockSpec((D,), lambda i: (0,)),
            pl.BlockSpec((D,), lambda i: (0,)),
            pl.BlockSpec((D_LAT, D), lambda i: (0, 0)),
            pl.BlockSpec((D_LAT,), lambda i: (0,)),
        ],
        out_specs=[
            pl.BlockSpec((G, D_LAT), lambda i: (0, 0)),
            pl.BlockSpec((G, D), lambda i: (0, 0)),
        ],
        out_shape=[
            jax.ShapeDtypeStruct((G, D_LAT), jnp.float32),
            jax.ShapeDtypeStruct((G, D), jnp.float32),
        ],
    )(numer, denom, h2, aux, b, batch,
      bn_gamma, bn_beta, bn_rm, bn_rv, fc_W, fc_b)
    return out


def kernel(x, edge_index, batch, W0, a_src0, a_dst0, b0, W1, a_src1, a_dst1,
           b1, W2, a_src2, a_dst2, b2, bn_gamma, bn_beta, bn_rm, bn_rv,
           fc_W, fc_b):
    src3d = edge_index[0].reshape(NS, CPW, CH)
    dst3d = edge_index[1].reshape(NS, CPW, CH)
    xp = jnp.pad(x, ((0, NP_ - N), (0, 0)))
    batch_p = jnp.pad(batch, (0, NP_ - N), constant_values=G)

    ws = [(W0, a_src0, a_dst0, b0.reshape(1, D)),
          (W1, a_src1, a_dst1, b1.reshape(1, D)),
          (W2, a_src2, a_dst2, b2.reshape(1, D))]

    h2, aux = _tc_entry(xp, ws[0][0], ws[0][1], ws[0][2])
    for l in (1, 2):
        numer, denom = _sc_edge_pass(src3d, dst3d,
                                     h2.reshape(NC * NP_, DH), aux)
        h2, aux = _tc_mid(numer, denom, h2, aux, ws[l - 1][3],
                          ws[l][0], ws[l][1], ws[l][2])
    numer, denom = _sc_edge_pass(src3d, dst3d, h2.reshape(NC * NP_, DH), aux)

    return _tc_final(numer, denom, h2, aux, ws[2][3], batch_p,
                     bn_gamma, bn_beta, bn_rm, bn_rv, fc_W, fc_b)


# speedup vs baseline: 1.0241x; 1.0059x over previous
import jax
import jax.numpy as jnp
from jax import lax
from jax.experimental import pallas as pl
from jax.experimental.pallas import tpu as pltpu
from jax.experimental.pallas import tpu_sc as plsc

N = 10000
E = 320000
G = 64
D = 128
DH = 64
D_LAT = 64

NP_ = 10240
NC = 2
NS = 16
CH = 80
EPW = E // NS
CPW = EPW // CH
TPN = NP_ // NS
NB = 4

RB = 2048
NBLK = NP_ // RB


def _sc_edge_body(src_hbm, dst_hbm, h2_hbm, aux_hbm,
                  numer_out, denom_out,
                  as_v, ad_v, src_v, dst_v, sadj_v, rows_v, w_v,
                  sem_g, sem_s, numer_s, denom_s):
    c = lax.axis_index("c")
    s = lax.axis_index("s")

    pltpu.sync_copy(aux_hbm.at[0], as_v)
    pltpu.sync_copy(aux_hbm.at[1], ad_v)
    pltpu.sync_copy(src_hbm.at[s], src_v)
    pltpu.sync_copy(dst_hbm.at[s], dst_v)

    hbase = c * NP_

    def compute_w(jj, b):
        for k in range(CH // 16):
            si = src_v[jj, pl.ds(k * 16, 16)]
            di = dst_v[jj, pl.ds(k * 16, 16)]
            sadj_v[b, pl.ds(k * 16, 16)] = si + hbase
            z = plsc.load_gather(as_v, [si]) + plsc.load_gather(ad_v, [di])
            z = jnp.where(z >= 0.0, z, 0.2 * z)
            w_v[b, pl.ds(k * 16, 16)] = jnp.exp(z)

    def gather_desc(b):
        return pltpu.make_async_copy(h2_hbm.at[sadj_v.at[b]], rows_v.at[b],
                                     sem_g.at[b])

    def scatter_descs(jj, b):
        return (pltpu.make_async_copy(rows_v.at[b], numer_s.at[dst_v.at[jj]],
                                      sem_s.at[b]),
                pltpu.make_async_copy(w_v.at[b], denom_s.at[dst_v.at[jj]],
                                      sem_s.at[b]))

    for b in range(NB - 1):
        compute_w(b, b)
        gather_desc(b).start()

    zeros16 = jnp.zeros((16,), jnp.float32)

    def _zrow(r, _):
        for cc in range(DH // 16):
            rows_v[NB - 1, r, pl.ds(cc * 16, 16)] = zeros16
        return 0

    lax.fori_loop(0, CH, _zrow, 0)
    for k in range(CH // 16):
        w_v[NB - 1, pl.ds(k * 16, 16)] = zeros16
    for k in range(TPN // CH):
        pltpu.sync_copy(rows_v.at[NB - 1],
                        numer_s.at[pl.ds(s * TPN + k * CH, CH)])
        pltpu.sync_copy(w_v.at[NB - 1],
                        denom_s.at[pl.ds(s * TPN + k * CH, CH)])
    plsc.subcore_barrier()

    def chunk_body(j, _):
        p = lax.rem(j, NB)
        r = lax.rem(j + NB - 1, NB)

        @pl.when(j + NB - 1 < CPW)
        def _():
            @pl.when(j >= 1)
            def _():
                for d in scatter_descs(j - 1, r):
                    d.wait()
            compute_w(j + NB - 1, r)
            gather_desc(r).start()

        gather_desc(p).wait()

        for g in range(CH // 16):
            w16 = w_v[p, pl.ds(g * 16, 16)]
            for i in range(16):
                rr = g * 16 + i
                ws = w16[i]
                for cc in range(DH // 16):
                    rows_v[p, rr, pl.ds(cc * 16, 16)] = (
                        rows_v[p, rr, pl.ds(cc * 16, 16)] * ws)

        pltpu.async_copy(rows_v.at[p], numer_s.at[dst_v.at[j]], sem_s.at[p],
                         add=True)
        pltpu.async_copy(w_v.at[p], denom_s.at[dst_v.at[j]], sem_s.at[p],
                         add=True)
        return 0

    lax.fori_loop(0, CPW, chunk_body, 0)
    for j in range(CPW - NB, CPW):
        for d in scatter_descs(j, j % NB):
            d.wait()
    plsc.subcore_barrier()

    pltpu.sync_copy(numer_s.at[pl.ds(s * TPN, TPN)],
                    numer_out.at[c, pl.ds(s * TPN, TPN)])
    pltpu.sync_copy(denom_s.at[pl.ds(s * TPN, TPN)],
                    denom_out.at[c, pl.ds(s * TPN, TPN)])


def _sc_edge_pass(src3d, dst3d, h2, aux):
    numer = jax.ShapeDtypeStruct((NC, NP_, DH), jnp.float32)
    denom = jax.ShapeDtypeStruct((NC, NP_), jnp.float32)
    f = pl.kernel(
        _sc_edge_body,
        out_type=(numer, denom),
        mesh=plsc.VectorSubcoreMesh(core_axis_name="c", subcore_axis_name="s",
                                    num_cores=NC, num_subcores=NS),
        compiler_params=pltpu.CompilerParams(needs_layout_passes=False,
                                             use_tc_tiling_on_sc=False),
        scratch_types=[
            pltpu.VMEM((NP_,), jnp.float32),
            pltpu.VMEM((NP_,), jnp.float32),
            pltpu.VMEM((CPW, CH), jnp.int32),
            pltpu.VMEM((CPW, CH), jnp.int32),
            pltpu.VMEM((NB, CH), jnp.int32),
            pltpu.VMEM((NB, CH, DH), jnp.float32),
            pltpu.VMEM((NB, CH), jnp.float32),
            pltpu.SemaphoreType.DMA((NB,)),
            pltpu.SemaphoreType.DMA((NB,)),
            pltpu.VMEM_SHARED((NP_, DH), jnp.float32),
            pltpu.VMEM_SHARED((NP_,), jnp.float32),
        ],
    )
    return f(src3d, dst3d, h2, aux)


def _split(h):
    return jnp.stack([h[:, :DH], h[:, DH:]], axis=0)


def _aux_of(h, a_src_ref, a_dst_ref):
    a2 = jnp.concatenate([a_src_ref[...][:, None], a_dst_ref[...][:, None]],
                         axis=1)
    return jnp.dot(h, a2, preferred_element_type=jnp.float32).T


def _tc_entry_body(x_ref, w_ref, as_ref, ad_ref, h2_ref, aux_ref):
    h = jnp.dot(x_ref[...], w_ref[...], preferred_element_type=jnp.float32)
    h2_ref[...] = _split(h)
    aux_ref[...] = _aux_of(h, as_ref, ad_ref)


def _tc_entry(x, w, a_src, a_dst):
    return pl.pallas_call(
        _tc_entry_body,
        grid=(NBLK,),
        in_specs=[
            pl.BlockSpec((RB, D), lambda i: (i, 0)),
            pl.BlockSpec((D, D), lambda i: (0, 0)),
            pl.BlockSpec((D,), lambda i: (0,)),
            pl.BlockSpec((D,), lambda i: (0,)),
        ],
        out_specs=[
            pl.BlockSpec((NC, RB, DH), lambda i: (0, i, 0)),
            pl.BlockSpec((2, RB), lambda i: (0, i)),
        ],
        out_shape=[
            jax.ShapeDtypeStruct((NC, NP_, DH), jnp.float32),
            jax.ShapeDtypeStruct((2, NP_), jnp.float32),
        ],
    )(x, w, a_src, a_dst)


def _epilogue_block(numer_ref, denom_ref, h2_ref, aux_ref, b_ref):
    aux = aux_ref[...]
    z = (aux[0, :] + aux[1, :])[:, None]
    wself = jnp.exp(jnp.where(z >= 0.0, z, 0.2 * z))
    h = jnp.concatenate([h2_ref[0], h2_ref[1]], axis=-1)
    numer = jnp.concatenate([numer_ref[0], numer_ref[1]], axis=-1) + wself * h
    denom = denom_ref[0, :] + wself[:, 0] + 1e-16
    return numer / denom[:, None] + b_ref[...]


def _tc_mid_body(numer_ref, denom_ref, h2_ref, aux_ref, b_ref, w_ref,
                 as_ref, ad_ref, h2o_ref, aux2_ref):
    x = _epilogue_block(numer_ref, denom_ref, h2_ref, aux_ref, b_ref)
    h2 = jnp.dot(x, w_ref[...], preferred_element_type=jnp.float32)
    h2o_ref[...] = _split(h2)
    aux2_ref[...] = _aux_of(h2, as_ref, ad_ref)


def _tc_mid(numer, denom, h2, aux, b, w, a_src, a_dst):
    return pl.pallas_call(
        _tc_mid_body,
        grid=(NBLK,),
        in_specs=[
            pl.BlockSpec((NC, RB, DH), lambda i: (0, i, 0)),
            pl.BlockSpec((NC, RB), lambda i: (0, i)),
            pl.BlockSpec((NC, RB, DH), lambda i: (0, i, 0)),
            pl.BlockSpec((2, RB), lambda i: (0, i)),
            pl.BlockSpec((1, D), lambda i: (0, 0)),
            pl.BlockSpec((D, D), lambda i: (0, 0)),
            pl.BlockSpec((D,), lambda i: (0,)),
            pl.BlockSpec((D,), lambda i: (0,)),
        ],
        out_specs=[
            pl.BlockSpec((NC, RB, DH), lambda i: (0, i, 0)),
            pl.BlockSpec((2, RB), lambda i: (0, i)),
        ],
        out_shape=[
            jax.ShapeDtypeStruct((NC, NP_, DH), jnp.float32),
            jax.ShapeDtypeStruct((2, NP_), jnp.float32),
        ],
    )(numer, denom, h2, aux, b, w, a_src, a_dst)


def _tc_final_body(numer_ref, denom_ref, h2_ref, aux_ref, b_ref, batch_ref,
                   bn_gamma_ref, bn_beta_ref, bn_rm_ref, bn_rv_ref,
                   fcw_ref, fcb_ref, out_ref, pooled_ref):
    i = pl.program_id(0)
    x = _epilogue_block(numer_ref, denom_ref, h2_ref, aux_ref, b_ref)
    onehot = (lax.broadcasted_iota(jnp.int32, (G, RB), 0)
              == batch_ref[...][None, :]).astype(jnp.float32)
    part = jnp.dot(onehot, x, preferred_element_type=jnp.float32)

    @pl.when(i == 0)
    def _():
        pooled_ref[...] = jnp.zeros_like(pooled_ref)

    pooled_ref[...] += part

    @pl.when(i == NBLK - 1)
    def _():
        bn_scale = (bn_gamma_ref[...]
                    / jnp.sqrt(bn_rv_ref[...] + 1e-5))[None, :]
        bn_shift = (bn_beta_ref[...] - bn_rm_ref[...]
                    / jnp.sqrt(bn_rv_ref[...] + 1e-5)
                    * bn_gamma_ref[...])[None, :]
        normed = pooled_ref[...] * bn_scale + bn_shift
        out_ref[...] = lax.dot_general(
            normed, fcw_ref[...], (((1,), (1,)), ((), ())),
            preferred_element_type=jnp.float32) + fcb_ref[...][None, :]


def _tc_final(numer, denom, h2, aux, b, batch,
              bn_gamma, bn_beta, bn_rm, bn_rv, fc_W, fc_b):
    out, _ = pl.pallas_call(
        _tc_final_body,
        grid=(NBLK,),
        in_specs=[
            pl.BlockSpec((NC, RB, DH), lambda i: (0, i, 0)),
            pl.BlockSpec((NC, RB), lambda i: (0, i)),
            pl.BlockSpec((NC, RB, DH), lambda i: (0, i, 0)),
            pl.BlockSpec((2, RB), lambda i: (0, i)),
            pl.BlockSpec((1, D), lambda i: (0, 0)),
            pl.BlockSpec((RB,), lambda i: (i,)),
            pl.BlockSpec((D,), lambda i: (0,)),
            pl.BlockSpec((D,), lambda i: (0,)),
            pl.BlockSpec((D,), lambda i: (0,)),
            pl.BlockSpec((D,), lambda i: (0,)),
            pl.BlockSpec((D_LAT, D), lambda i: (0, 0)),
            pl.BlockSpec((D_LAT,), lambda i: (0,)),
        ],
        out_specs=[
            pl.BlockSpec((G, D_LAT), lambda i: (0, 0)),
            pl.BlockSpec((G, D), lambda i: (0, 0)),
        ],
        out_shape=[
            jax.ShapeDtypeStruct((G, D_LAT), jnp.float32),
            jax.ShapeDtypeStruct((G, D), jnp.float32),
        ],
    )(numer, denom, h2, aux, b, batch,
      bn_gamma, bn_beta, bn_rm, bn_rv, fc_W, fc_b)
    return out


def kernel(x, edge_index, batch, W0, a_src0, a_dst0, b0, W1, a_src1, a_dst1,
           b1, W2, a_src2, a_dst2, b2, bn_gamma, bn_beta, bn_rm, bn_rv,
           fc_W, fc_b):
    src3d = edge_index[0].reshape(NS, CPW, CH)
    dst3d = edge_index[1].reshape(NS, CPW, CH)
    xp = jnp.pad(x, ((0, NP_ - N), (0, 0)))
    batch_p = jnp.pad(batch, (0, NP_ - N), constant_values=G)

    ws = [(W0, a_src0, a_dst0, b0.reshape(1, D)),
          (W1, a_src1, a_dst1, b1.reshape(1, D)),
          (W2, a_src2, a_dst2, b2.reshape(1, D))]

    h2, aux = _tc_entry(xp, ws[0][0], ws[0][1], ws[0][2])
    for l in (1, 2):
        numer, denom = _sc_edge_pass(src3d, dst3d,
                                     h2.reshape(NC * NP_, DH), aux)
        h2, aux = _tc_mid(numer, denom, h2, aux, ws[l - 1][3],
                          ws[l][0], ws[l][1], ws[l][2])
    numer, denom = _sc_edge_pass(src3d, dst3d, h2.reshape(NC * NP_, DH), aux)

    return _tc_final(numer, denom, h2, aux, ws[2][3], batch_p,
                     bn_gamma, bn_beta, bn_rm, bn_rv, fc_W, fc_b)
